# baseline (speedup 1.0000x reference)
import functools

import jax
import jax.numpy as jnp
from jax import lax
from jax.experimental import pallas as pl
from jax.experimental.pallas import tpu as pltpu
from jax.experimental.pallas import tpu_sc as plsc

EPS = 1e-12


def _make_sc_gather(V, D, B):
    info = plsc.get_sparse_core_info()
    NC, NS = info.num_cores, info.num_subcores
    NW = NC * NS
    assert B % NW == 0
    b_per_w = B // NW
    C = min(b_per_w, 32)
    assert b_per_w % C == 0
    n_chunks = b_per_w // C
    mesh = plsc.VectorSubcoreMesh(core_axis_name="c", subcore_axis_name="s")

    @functools.partial(
        pl.kernel,
        mesh=mesh,
        out_type=jax.ShapeDtypeStruct((B, D), jnp.float32),
        scratch_types=[
            pltpu.VMEM((b_per_w,), jnp.int32),
            pltpu.VMEM((C, D), jnp.float32),
            pltpu.VMEM((C, D), jnp.float32),
            pltpu.SemaphoreType.DMA,
            pltpu.SemaphoreType.DMA,
            pltpu.SemaphoreType.DMA,
        ],
    )
    def sc_gather(table_hbm, idx_hbm, out_hbm, idx_v, rows0, rows1, gsem, ws0, ws1):
        wid = lax.axis_index("s") * NC + lax.axis_index("c")
        base = wid * b_per_w
        pltpu.sync_copy(idx_hbm.at[pl.ds(base, b_per_w)], idx_v)
        rows = (rows0, rows1)
        wsem = (ws0, ws1)
        writes = [None, None]
        for c in range(n_chunks):
            s = c % 2
            if writes[s] is not None:
                writes[s].wait()
            pltpu.async_copy(
                table_hbm.at[idx_v.at[pl.ds(c * C, C)]], rows[s], gsem
            ).wait()
            w = pltpu.make_async_copy(
                rows[s], out_hbm.at[pl.ds(base + c * C, C)], wsem[s]
            )
            w.start()
            writes[s] = w
        for w in writes:
            if w is not None:
                w.wait()

    return sc_gather


def _tc_add_ln_body(g_ref, p_ref, gamma_ref, beta_ref, o_ref):
    x = g_ref[...] + p_ref[...][None, :, :]
    mean = jnp.mean(x, axis=-1, keepdims=True)
    xc = x - mean
    var = jnp.mean(xc * xc, axis=-1, keepdims=True)
    xhat = xc * lax.rsqrt(var + EPS)
    o_ref[...] = xhat * gamma_ref[...] + beta_ref[...]


def _tc_add_ln(gathered3, pos_emb, gamma, beta, R=2048):
    Bt, S, D = gathered3.shape
    pos_blocks = S // R
    return pl.pallas_call(
        _tc_add_ln_body,
        grid=(pos_blocks, Bt),
        in_specs=[
            pl.BlockSpec((1, R, D), lambda j, b: (b, j, 0)),
            pl.BlockSpec((R, D), lambda j, b: (j, 0)),
            pl.BlockSpec((1, D), lambda j, b: (0, 0)),
            pl.BlockSpec((1, D), lambda j, b: (0, 0)),
        ],
        out_specs=pl.BlockSpec((1, R, D), lambda j, b: (b, j, 0)),
        out_shape=jax.ShapeDtypeStruct((Bt, S, D), jnp.float32),
    )(gathered3, pos_emb, gamma.reshape(1, D), beta.reshape(1, D))


def kernel(input_ids, word_emb, pos_emb, ln_gamma, ln_beta):
    Bt, S = input_ids.shape
    V, D = word_emb.shape
    ids = input_ids.reshape(-1).astype(jnp.int32)
    gathered = _make_sc_gather(V, D, Bt * S)(word_emb, ids)
    return _tc_add_ln(gathered.reshape(Bt, S, D), pos_emb, ln_gamma, ln_beta)


# speedup vs baseline: 1.0097x; 1.0097x over previous
import functools

import jax
import jax.numpy as jnp
from jax import lax
from jax.experimental import pallas as pl
from jax.experimental.pallas import tpu as pltpu
from jax.experimental.pallas import tpu_sc as plsc

EPS = 1e-12


def _make_sc_gather(V, D, B):
    info = plsc.get_sparse_core_info()
    NC, NS = info.num_cores, info.num_subcores
    NW = NC * NS
    assert B % NW == 0
    b_per_w = B // NW
    C = min(b_per_w, 32)
    assert b_per_w % C == 0
    n_chunks = b_per_w // C
    mesh = plsc.VectorSubcoreMesh(core_axis_name="c", subcore_axis_name="s")

    @functools.partial(
        pl.kernel,
        mesh=mesh,
        out_type=jax.ShapeDtypeStruct((B, D), jnp.float32),
        scratch_types=[
            pltpu.VMEM((b_per_w,), jnp.int32),
            pltpu.VMEM((C, D), jnp.float32),
            pltpu.VMEM((C, D), jnp.float32),
            pltpu.SemaphoreType.DMA,
            pltpu.SemaphoreType.DMA,
            pltpu.SemaphoreType.DMA,
        ],
    )
    def sc_gather(table_hbm, idx_hbm, out_hbm, idx_v, rows0, rows1, gsem, ws0, ws1):
        wid = lax.axis_index("s") * NC + lax.axis_index("c")
        base = wid * b_per_w
        pltpu.sync_copy(idx_hbm.at[pl.ds(base, b_per_w)], idx_v)
        rows = (rows0, rows1)
        wsem = (ws0, ws1)
        writes = [None, None]
        for c in range(n_chunks):
            s = c % 2
            if writes[s] is not None:
                writes[s].wait()
            pltpu.async_copy(
                table_hbm.at[idx_v.at[pl.ds(c * C, C)]], rows[s], gsem
            ).wait()
            w = pltpu.make_async_copy(
                rows[s], out_hbm.at[pl.ds(base + c * C, C)], wsem[s]
            )
            w.start()
            writes[s] = w
        for w in writes:
            if w is not None:
                w.wait()

    return sc_gather


def _tc_slice_body(g_ref, p_ref, gamma_ref, beta_ref, o_ref):
    x = g_ref[...] + p_ref[...][None, :, :]
    mean = jnp.mean(x, axis=-1, keepdims=True)
    xc = x - mean
    var = jnp.mean(xc * xc, axis=-1, keepdims=True)
    xhat = xc * lax.rsqrt(var + EPS)
    o_ref[...] = xhat * gamma_ref[...] + beta_ref[...]


def _tc_add_ln_slice(buf, b0, Bt, gathered3, pos_emb, gamma, beta):
    nb, S, D = gathered3.shape
    R = 2048
    first = buf is None
    data_specs = [
        pl.BlockSpec((1, R, D), lambda b: (b, 0, 0)),
        pl.BlockSpec((R, D), lambda b: (0, 0)),
        pl.BlockSpec((1, D), lambda b: (0, 0)),
        pl.BlockSpec((1, D), lambda b: (0, 0)),
    ]
    in_specs = data_specs if first else [pl.BlockSpec(memory_space=pl.ANY)] + data_specs
    body = _tc_slice_body if first else (lambda d, *a: _tc_slice_body(*a))
    args = () if first else (buf,)
    return pl.pallas_call(
        body,
        grid=(nb,),
        in_specs=in_specs,
        out_specs=pl.BlockSpec((1, R, D), lambda b: (b0 + b, 0, 0)),
        out_shape=jax.ShapeDtypeStruct((Bt, S, D), jnp.float32),
        input_output_aliases={} if first else {0: 0},
    )(*args, gathered3, pos_emb, gamma.reshape(1, D), beta.reshape(1, D))


def kernel(input_ids, word_emb, pos_emb, ln_gamma, ln_beta):
    Bt, S = input_ids.shape
    V, D = word_emb.shape
    ids = input_ids.astype(jnp.int32)
    nslices, nb = 2, Bt // 2
    sc_gather = _make_sc_gather(V, D, nb * S)
    gathered = [
        sc_gather(word_emb, ids[k * nb:(k + 1) * nb].reshape(-1)).reshape(nb, S, D)
        for k in range(nslices)
    ]
    buf = None
    for k in range(nslices):
        buf = _tc_add_ln_slice(buf, k * nb, Bt, gathered[k], pos_emb, ln_gamma, ln_beta)
    return buf
